# baseline (speedup 1.0000x reference)
import functools

import jax
import jax.numpy as jnp
from jax import lax
from jax.experimental import pallas as pl
from jax.experimental.pallas import tpu as pltpu
from jax.experimental.pallas import tpu_sc as plsc

B, T, C = 1, 2048, 2048
E, TOPK = 16, 2
H = 1024
HS = 2048
N = B * T
NA = N * TOPK
TM = 128
W = 48
P = W * TM

NC, NS = 2, 16
NW = NC * NS


CI = C // 2


@functools.lru_cache(maxsize=None)
def _sc_mesh():
    return plsc.VectorSubcoreMesh(core_axis_name="c", subcore_axis_name="s")


def _bf_to_i32(a):
    return lax.bitcast_convert_type(
        a.reshape(*a.shape[:-1], a.shape[-1] // 2, 2), jnp.int32)


def _i32_to_bf(a):
    return lax.bitcast_convert_type(a, jnp.bfloat16).reshape(
        *a.shape[:-1], a.shape[-1] * 2)


def _pipelined_gathers(src_hbm, idx_v, out_hbm, out_base, bufs, sems, nch, ch):
    nbuf = len(bufs)
    gathers = [None] * nch
    writes = [None] * nch
    gathers[0] = pltpu.async_copy(
        src_hbm.at[idx_v.at[pl.ds(0, ch)]], bufs[0], sems[0])
    for ci in range(nch):
        k = ci % nbuf
        if ci + 1 < nch:
            kn = (ci + 1) % nbuf
            if ci + 1 >= nbuf:
                writes[ci + 1 - nbuf].wait()
            gathers[ci + 1] = pltpu.async_copy(
                src_hbm.at[idx_v.at[pl.ds((ci + 1) * ch, ch)]],
                bufs[kn], sems[kn])
        gathers[ci].wait()
        writes[ci] = pltpu.async_copy(
            bufs[k], out_hbm.at[pl.ds(out_base + ci * ch, ch)], sems[k])
    for ci in range(max(0, nch - nbuf), nch):
        writes[ci].wait()


def _router_body(x_ref, gw_ref, gb_ref, idx_ref, w_ref):
    s = jax.nn.sigmoid(
        jnp.dot(x_ref[...], gw_ref[...], preferred_element_type=jnp.float32))
    sb = s + gb_ref[...]
    tm = s.shape[0]
    iota = lax.broadcasted_iota(jnp.int32, (tm, E), 1)
    m1 = jnp.max(sb, axis=1, keepdims=True)
    i1 = jnp.min(jnp.where(sb == m1, iota, E), axis=1, keepdims=True)
    sb2 = jnp.where(iota == i1, -jnp.inf, sb)
    m2 = jnp.max(sb2, axis=1, keepdims=True)
    i2 = jnp.min(jnp.where(sb2 == m2, iota, E), axis=1, keepdims=True)
    w1 = jnp.sum(jnp.where(iota == i1, s, 0.0), axis=1, keepdims=True)
    w2 = jnp.sum(jnp.where(iota == i2, s, 0.0), axis=1, keepdims=True)
    den = w1 + w2
    idx_ref[...] = jnp.where(iota == 0, i1, jnp.where(iota == 1, i2, 0))
    w_ref[...] = jnp.where(iota == 0, w1 / den,
                           jnp.where(iota == 1, w2 / den, 0.0))


def _router(x_flat, gate_w, gate_bias):
    tm = 256
    return pl.pallas_call(
        _router_body,
        grid=(N // tm,),
        in_specs=[
            pl.BlockSpec((tm, C), lambda t: (t, 0)),
            pl.BlockSpec((C, E), lambda t: (0, 0)),
            pl.BlockSpec((1, E), lambda t: (0, 0)),
        ],
        out_specs=[
            pl.BlockSpec((tm, E), lambda t: (t, 0)),
            pl.BlockSpec((tm, E), lambda t: (t, 0)),
        ],
        out_shape=[
            jax.ShapeDtypeStruct((N, E), jnp.int32),
            jax.ShapeDtypeStruct((N, E), jnp.float32),
        ],
    )(x_flat, gate_w, gate_bias.reshape(1, E))


_D_RPW = P // NW
_D_CH = 32
_D_NCH = _D_RPW // _D_CH


def _dispatch_body(x_hbm, tok_hbm, out_hbm, idx_v, b0, b1, s0, s1):
    wid = lax.axis_index("s") * NC + lax.axis_index("c")
    base = wid * _D_RPW
    pltpu.sync_copy(tok_hbm.at[pl.ds(base, _D_RPW)], idx_v)
    _pipelined_gathers(x_hbm, idx_v, out_hbm, base, (b0, b1), (s0, s1),
                       _D_NCH, _D_CH)


@functools.lru_cache(maxsize=None)
def _dispatch_kernel():
    return pl.kernel(
        _dispatch_body,
        out_type=jax.ShapeDtypeStruct((P, CI), jnp.int32),
        mesh=_sc_mesh(),
        scratch_types=[
            pltpu.VMEM((_D_RPW,), jnp.int32),
            pltpu.VMEM((_D_CH, CI), jnp.int32),
            pltpu.VMEM((_D_CH, CI), jnp.int32),
            pltpu.SemaphoreType.DMA,
            pltpu.SemaphoreType.DMA,
        ],
    )


def _dispatch(x_bf, tok1d):
    return _dispatch_kernel()(x_bf, tok1d)


def _gemm_body(e_ref, v_ref, x_ref, up_ref, dn_ref, w_ref, o_ref):
    i = pl.program_id(0)

    @pl.when(v_ref[i] == 1)
    def _():
        yg = jnp.dot(x_ref[...].astype(jnp.float32), up_ref[0],
                     preferred_element_type=jnp.float32)
        gv = yg[:, :H]
        uv = yg[:, H:]
        h = gv * jax.nn.sigmoid(gv) * uv
        oe = jnp.dot(h, dn_ref[0], preferred_element_type=jnp.float32)
        o_ref[...] = (oe * w_ref[:, 0:1]).astype(jnp.bfloat16)


def _grouped_gemm(x_perm, up_w, down_w, wpad2d, e_for_tile, valid):
    grid_spec = pltpu.PrefetchScalarGridSpec(
        num_scalar_prefetch=2,
        grid=(W,),
        in_specs=[
            pl.BlockSpec((TM, C), lambda i, e, v: (i, 0)),
            pl.BlockSpec((1, C, 2 * H), lambda i, e, v: (e[i], 0, 0)),
            pl.BlockSpec((1, H, C), lambda i, e, v: (e[i], 0, 0)),
            pl.BlockSpec((TM, 128), lambda i, e, v: (i, 0)),
        ],
        out_specs=pl.BlockSpec((TM, C), lambda i, e, v: (i, 0)),
    )
    return pl.pallas_call(
        _gemm_body,
        grid_spec=grid_spec,
        out_shape=jax.ShapeDtypeStruct((P, C), jnp.bfloat16),
    )(e_for_tile, valid, x_perm, up_w, down_w, wpad2d)


_C_RPW = N // NW
_C_CH = 16
_C_NCH = _C_RPW // _C_CH


def _combine_body(src_hbm, p0_hbm, p1_hbm, g0_hbm, g1_hbm, i0_v, i1_v,
                  b00, b01, b10, b11, s00, s01, s10, s11):
    wid = lax.axis_index("s") * NC + lax.axis_index("c")
    base = wid * _C_RPW
    pltpu.sync_copy(p0_hbm.at[pl.ds(base, _C_RPW)], i0_v)
    pltpu.sync_copy(p1_hbm.at[pl.ds(base, _C_RPW)], i1_v)
    _pipelined_gathers(src_hbm, i0_v, g0_hbm, base, (b00, b01), (s00, s01),
                       _C_NCH, _C_CH)
    _pipelined_gathers(src_hbm, i1_v, g1_hbm, base, (b10, b11), (s10, s11),
                       _C_NCH, _C_CH)


@functools.lru_cache(maxsize=None)
def _combine_kernel():
    return pl.kernel(
        _combine_body,
        out_type=(
            jax.ShapeDtypeStruct((N, CI), jnp.int32),
            jax.ShapeDtypeStruct((N, CI), jnp.int32),
        ),
        mesh=_sc_mesh(),
        scratch_types=[
            pltpu.VMEM((_C_RPW,), jnp.int32),
            pltpu.VMEM((_C_RPW,), jnp.int32),
            pltpu.VMEM((_C_CH, CI), jnp.int32),
            pltpu.VMEM((_C_CH, CI), jnp.int32),
            pltpu.VMEM((_C_CH, CI), jnp.int32),
            pltpu.VMEM((_C_CH, CI), jnp.int32),
            pltpu.SemaphoreType.DMA,
            pltpu.SemaphoreType.DMA,
            pltpu.SemaphoreType.DMA,
            pltpu.SemaphoreType.DMA,
        ],
    )


def _combine(out_perm, pos0, pos1):
    return _combine_kernel()(out_perm, pos0, pos1)


def _shared_body(x_ref, sg_ref, sd_ref, g0_ref, g1_ref, o_ref):
    yg = jnp.dot(x_ref[...], sg_ref[...], preferred_element_type=jnp.float32)
    y = yg[:, :HS]
    gate = yg[:, HS:]
    h = gate * jax.nn.sigmoid(gate) * y
    o_ref[...] = (jnp.dot(h, sd_ref[...], preferred_element_type=jnp.float32)
                  + g0_ref[...].astype(jnp.float32)
                  + g1_ref[...].astype(jnp.float32))


def _shared(x_flat, sgw, sdw, g0, g1):
    tm = 128
    return pl.pallas_call(
        _shared_body,
        grid=(N // tm,),
        in_specs=[
            pl.BlockSpec((tm, C), lambda t: (t, 0)),
            pl.BlockSpec((C, 2 * HS), lambda t: (0, 0)),
            pl.BlockSpec((HS, C), lambda t: (0, 0)),
            pl.BlockSpec((tm, C), lambda t: (t, 0)),
            pl.BlockSpec((tm, C), lambda t: (t, 0)),
        ],
        out_specs=pl.BlockSpec((tm, C), lambda t: (t, 0)),
        out_shape=jax.ShapeDtypeStruct((N, C), jnp.float32),
    )(x_flat, sgw, sdw, g0, g1)


def kernel(x, gate_w, gate_bias, shared_gate_w, shared_down_w, up_w, down_w):
    x_flat = x.reshape(N, C)
    x_bf = x_flat.astype(jnp.bfloat16)

    idx_out, w_out = _router(x_flat, gate_w, gate_bias)
    eN = idx_out[:, :TOPK].reshape(-1)
    wN = w_out[:, :TOPK].reshape(-1)

    onehot = (eN[:, None] == jnp.arange(E, dtype=jnp.int32)[None, :])
    ranks = jnp.cumsum(onehot.astype(jnp.int32), axis=0)
    counts = ranks[-1]
    rank = jnp.take_along_axis(ranks, eN[:, None], axis=1)[:, 0] - 1
    padded = ((counts + TM - 1) // TM) * TM
    pad_off = jnp.concatenate(
        [jnp.zeros((1,), jnp.int32), jnp.cumsum(padded).astype(jnp.int32)])
    ppos = pad_off[eN] + rank
    tok = (jnp.arange(NA, dtype=jnp.int32) // TOPK)
    tok_pad = (jnp.arange(P, dtype=jnp.int32) % N).at[ppos].set(tok)
    w_pad = jnp.zeros((P,), jnp.float32).at[ppos].set(wN)
    wpad2d = jnp.broadcast_to(w_pad[:, None], (P, 128))
    tile_off = pad_off // TM
    t_ar = jnp.arange(W, dtype=jnp.int32)
    e_for_tile = jnp.minimum(
        jnp.sum((t_ar[:, None] >= tile_off[None, 1:]).astype(jnp.int32),
                axis=1), E - 1).astype(jnp.int32)
    valid = (t_ar < tile_off[E]).astype(jnp.int32)
    pos0 = ppos[0::2]
    pos1 = ppos[1::2]

    x_perm = _i32_to_bf(_dispatch(_bf_to_i32(x_bf), tok_pad))

    out_perm = _grouped_gemm(x_perm, up_w, down_w, wpad2d, e_for_tile, valid)

    g0i, g1i = _combine(_bf_to_i32(out_perm), pos0, pos1)
    g0 = _i32_to_bf(g0i)
    g1 = _i32_to_bf(g1i)

    out = _shared(x_flat, shared_gate_w, shared_down_w, g0, g1)
    return out.reshape(B, T, C)


# speedup vs baseline: 2.9410x; 2.9410x over previous
import functools

import jax
import jax.numpy as jnp
from jax import lax
from jax.experimental import pallas as pl
from jax.experimental.pallas import tpu as pltpu
from jax.experimental.pallas import tpu_sc as plsc

B, T, C = 1, 2048, 2048
E, TOPK = 16, 2
H = 1024
HS = 2048
N = B * T
NA = N * TOPK
TM = 128
W = 48
P = W * TM
CH_ = C // 2

NC, NS = 2, 16
NW = NC * NS

_LOW = 0x0000FFFF
_HIGH = -65536


@functools.lru_cache(maxsize=None)
def _sc_mesh():
    return plsc.VectorSubcoreMesh(core_axis_name="c", subcore_axis_name="s")


def _pack_halves(left_f32, right_f32):
    lb = lax.bitcast_convert_type(
        left_f32.astype(jnp.bfloat16).astype(jnp.float32), jnp.int32)
    rb = lax.bitcast_convert_type(
        right_f32.astype(jnp.bfloat16).astype(jnp.float32), jnp.int32)
    return ((lb >> 16) & _LOW) | (rb & _HIGH)


def _unpack_halves(words_i32):
    left = lax.bitcast_convert_type(words_i32 << 16, jnp.float32)
    right = lax.bitcast_convert_type(words_i32 & _HIGH, jnp.float32)
    return left, right


def _pipelined_gathers(src_hbm, idx_v, out_hbm, out_base, bufs, sems, nch, ch):
    nbuf = len(bufs)
    gathers = [None] * nch
    writes = [None] * nch
    gathers[0] = pltpu.async_copy(
        src_hbm.at[idx_v.at[pl.ds(0, ch)]], bufs[0], sems[0])
    for ci in range(nch):
        k = ci % nbuf
        if ci + 1 < nch:
            kn = (ci + 1) % nbuf
            if ci + 1 >= nbuf:
                writes[ci + 1 - nbuf].wait()
            gathers[ci + 1] = pltpu.async_copy(
                src_hbm.at[idx_v.at[pl.ds((ci + 1) * ch, ch)]],
                bufs[kn], sems[kn])
        gathers[ci].wait()
        writes[ci] = pltpu.async_copy(
            bufs[k], out_hbm.at[pl.ds(out_base + ci * ch, ch)], sems[k])
    for ci in range(max(0, nch - nbuf), nch):
        writes[ci].wait()


def _router_body(x_ref, gw_ref, gb_ref, idx_ref, w_ref, xp_ref):
    x = x_ref[...]
    s = jax.nn.sigmoid(
        jnp.dot(x, gw_ref[...], preferred_element_type=jnp.float32))
    sb = s + gb_ref[...]
    tm = s.shape[0]
    iota = lax.broadcasted_iota(jnp.int32, (tm, E), 1)
    m1 = jnp.max(sb, axis=1, keepdims=True)
    i1 = jnp.min(jnp.where(sb == m1, iota, E), axis=1, keepdims=True)
    sb2 = jnp.where(iota == i1, -jnp.inf, sb)
    m2 = jnp.max(sb2, axis=1, keepdims=True)
    i2 = jnp.min(jnp.where(sb2 == m2, iota, E), axis=1, keepdims=True)
    w1 = jnp.sum(jnp.where(iota == i1, s, 0.0), axis=1, keepdims=True)
    w2 = jnp.sum(jnp.where(iota == i2, s, 0.0), axis=1, keepdims=True)
    den = w1 + w2
    idx_ref[...] = jnp.where(iota == 0, i1, jnp.where(iota == 1, i2, 0))
    w_ref[...] = jnp.where(iota == 0, w1 / den,
                           jnp.where(iota == 1, w2 / den, 0.0))
    xp_ref[...] = _pack_halves(x[:, :CH_], x[:, CH_:])


def _router(x_flat, gate_w, gate_bias):
    tm = 256
    return pl.pallas_call(
        _router_body,
        grid=(N // tm,),
        in_specs=[
            pl.BlockSpec((tm, C), lambda t: (t, 0)),
            pl.BlockSpec((C, E), lambda t: (0, 0)),
            pl.BlockSpec((1, E), lambda t: (0, 0)),
        ],
        out_specs=[
            pl.BlockSpec((tm, E), lambda t: (t, 0)),
            pl.BlockSpec((tm, E), lambda t: (t, 0)),
            pl.BlockSpec((tm, CH_), lambda t: (t, 0)),
        ],
        out_shape=[
            jax.ShapeDtypeStruct((N, E), jnp.int32),
            jax.ShapeDtypeStruct((N, E), jnp.float32),
            jax.ShapeDtypeStruct((N, CH_), jnp.int32),
        ],
    )(x_flat, gate_w, gate_bias.reshape(1, E))


_D_RPW = P // NW
_D_CH = 32
_D_NCH = _D_RPW // _D_CH


def _dispatch_body(x_hbm, tok_hbm, out_hbm, idx_v, b0, b1, s0, s1):
    wid = lax.axis_index("s") * NC + lax.axis_index("c")
    base = wid * _D_RPW
    pltpu.sync_copy(tok_hbm.at[pl.ds(base, _D_RPW)], idx_v)
    _pipelined_gathers(x_hbm, idx_v, out_hbm, base, (b0, b1), (s0, s1),
                       _D_NCH, _D_CH)


@functools.lru_cache(maxsize=None)
def _dispatch_kernel():
    return pl.kernel(
        _dispatch_body,
        out_type=jax.ShapeDtypeStruct((P, CH_), jnp.int32),
        mesh=_sc_mesh(),
        scratch_types=[
            pltpu.VMEM((_D_RPW,), jnp.int32),
            pltpu.VMEM((_D_CH, CH_), jnp.int32),
            pltpu.VMEM((_D_CH, CH_), jnp.int32),
            pltpu.SemaphoreType.DMA,
            pltpu.SemaphoreType.DMA,
        ],
    )


def _dispatch(x_packed, tok1d):
    return _dispatch_kernel()(x_packed, tok1d)


def _gemm_body(e_ref, v_ref, x_ref, uplo_ref, uphi_ref, dn_ref, w_ref, o_ref):
    i = pl.program_id(0)

    @pl.when(v_ref[i] == 1)
    def _():
        lo, hi = _unpack_halves(x_ref[...])
        yg = (jnp.dot(lo, uplo_ref[0], preferred_element_type=jnp.float32)
              + jnp.dot(hi, uphi_ref[0], preferred_element_type=jnp.float32))
        gv = yg[:, :H]
        uv = yg[:, H:]
        h = gv * jax.nn.sigmoid(gv) * uv
        oe = jnp.dot(h, dn_ref[0], preferred_element_type=jnp.float32)
        oe = oe * w_ref[:, 0:1]
        o_ref[...] = _pack_halves(oe[:, :CH_], oe[:, CH_:])


def _grouped_gemm(x_perm, up_w, down_w, wpad2d, e_for_tile, valid):
    grid_spec = pltpu.PrefetchScalarGridSpec(
        num_scalar_prefetch=2,
        grid=(W,),
        in_specs=[
            pl.BlockSpec((TM, CH_), lambda i, e, v: (i, 0)),
            pl.BlockSpec((1, CH_, 2 * H), lambda i, e, v: (e[i], 0, 0)),
            pl.BlockSpec((1, CH_, 2 * H), lambda i, e, v: (e[i], 1, 0)),
            pl.BlockSpec((1, H, C), lambda i, e, v: (e[i], 0, 0)),
            pl.BlockSpec((TM, 128), lambda i, e, v: (i, 0)),
        ],
        out_specs=pl.BlockSpec((TM, CH_), lambda i, e, v: (i, 0)),
    )
    return pl.pallas_call(
        _gemm_body,
        grid_spec=grid_spec,
        out_shape=jax.ShapeDtypeStruct((P, CH_), jnp.int32),
    )(e_for_tile, valid, x_perm, up_w, up_w, down_w, wpad2d)


_C_RPW = N // NW
_C_CH = 16
_C_NCH = _C_RPW // _C_CH


def _combine_body(src_hbm, p0_hbm, p1_hbm, g0_hbm, g1_hbm, i0_v, i1_v,
                  b00, b01, b10, b11, s00, s01, s10, s11):
    wid = lax.axis_index("s") * NC + lax.axis_index("c")
    base = wid * _C_RPW
    pltpu.sync_copy(p0_hbm.at[pl.ds(base, _C_RPW)], i0_v)
    pltpu.sync_copy(p1_hbm.at[pl.ds(base, _C_RPW)], i1_v)
    _pipelined_gathers(src_hbm, i0_v, g0_hbm, base, (b00, b01), (s00, s01),
                       _C_NCH, _C_CH)
    _pipelined_gathers(src_hbm, i1_v, g1_hbm, base, (b10, b11), (s10, s11),
                       _C_NCH, _C_CH)


@functools.lru_cache(maxsize=None)
def _combine_kernel():
    return pl.kernel(
        _combine_body,
        out_type=(
            jax.ShapeDtypeStruct((N, CH_), jnp.int32),
            jax.ShapeDtypeStruct((N, CH_), jnp.int32),
        ),
        mesh=_sc_mesh(),
        scratch_types=[
            pltpu.VMEM((_C_RPW,), jnp.int32),
            pltpu.VMEM((_C_RPW,), jnp.int32),
            pltpu.VMEM((_C_CH, CH_), jnp.int32),
            pltpu.VMEM((_C_CH, CH_), jnp.int32),
            pltpu.VMEM((_C_CH, CH_), jnp.int32),
            pltpu.VMEM((_C_CH, CH_), jnp.int32),
            pltpu.SemaphoreType.DMA,
            pltpu.SemaphoreType.DMA,
            pltpu.SemaphoreType.DMA,
            pltpu.SemaphoreType.DMA,
        ],
    )


def _combine(out_perm, pos0, pos1):
    return _combine_kernel()(out_perm, pos0, pos1)


def _shared_body(x_ref, sg_ref, sd_ref, g0_ref, g1_ref, o_ref):
    yg = jnp.dot(x_ref[...], sg_ref[...], preferred_element_type=jnp.float32)
    y = yg[:, :HS]
    gate = yg[:, HS:]
    h = gate * jax.nn.sigmoid(gate) * y
    base = jnp.dot(h, sd_ref[...], preferred_element_type=jnp.float32)
    g0l, g0r = _unpack_halves(g0_ref[...])
    g1l, g1r = _unpack_halves(g1_ref[...])
    o_ref[:, :CH_] = base[:, :CH_] + g0l + g1l
    o_ref[:, CH_:] = base[:, CH_:] + g0r + g1r


def _shared(x_flat, sgw, sdw, g0, g1):
    tm = 128
    return pl.pallas_call(
        _shared_body,
        grid=(N // tm,),
        in_specs=[
            pl.BlockSpec((tm, C), lambda t: (t, 0)),
            pl.BlockSpec((C, 2 * HS), lambda t: (0, 0)),
            pl.BlockSpec((HS, C), lambda t: (0, 0)),
            pl.BlockSpec((tm, CH_), lambda t: (t, 0)),
            pl.BlockSpec((tm, CH_), lambda t: (t, 0)),
        ],
        out_specs=pl.BlockSpec((tm, C), lambda t: (t, 0)),
        out_shape=jax.ShapeDtypeStruct((N, C), jnp.float32),
    )(x_flat, sgw, sdw, g0, g1)


def kernel(x, gate_w, gate_bias, shared_gate_w, shared_down_w, up_w, down_w):
    x_flat = x.reshape(N, C)

    idx_out, w_out, x_packed = _router(x_flat, gate_w, gate_bias)
    eN = idx_out[:, :TOPK].reshape(-1)
    wN = w_out[:, :TOPK].reshape(-1)

    onehot = (eN[:, None] == jnp.arange(E, dtype=jnp.int32)[None, :])
    ranks = jnp.cumsum(onehot.astype(jnp.int32), axis=0)
    counts = ranks[-1]
    rank = jnp.take_along_axis(ranks, eN[:, None], axis=1)[:, 0] - 1
    padded = ((counts + TM - 1) // TM) * TM
    pad_off = jnp.concatenate(
        [jnp.zeros((1,), jnp.int32), jnp.cumsum(padded).astype(jnp.int32)])
    ppos = pad_off[eN] + rank
    tok = (jnp.arange(NA, dtype=jnp.int32) // TOPK)
    tok_pad = (jnp.arange(P, dtype=jnp.int32) % N).at[ppos].set(tok)
    w_pad = jnp.zeros((P,), jnp.float32).at[ppos].set(wN)
    wpad2d = jnp.broadcast_to(w_pad[:, None], (P, 128))
    tile_off = pad_off // TM
    t_ar = jnp.arange(W, dtype=jnp.int32)
    e_for_tile = jnp.minimum(
        jnp.sum((t_ar[:, None] >= tile_off[None, 1:]).astype(jnp.int32),
                axis=1), E - 1).astype(jnp.int32)
    valid = (t_ar < tile_off[E]).astype(jnp.int32)
    pos0 = ppos[0::2]
    pos1 = ppos[1::2]

    x_perm = _dispatch(x_packed, tok_pad)

    out_perm = _grouped_gemm(x_perm, up_w, down_w, wpad2d, e_for_tile, valid)

    g0, g1 = _combine(out_perm, pos0, pos1)

    out = _shared(x_flat, shared_gate_w, shared_down_w, g0, g1)
    return out.reshape(B, T, C)


# speedup vs baseline: 3.0103x; 1.0236x over previous
import functools

import jax
import jax.numpy as jnp
from jax import lax
from jax.experimental import pallas as pl
from jax.experimental.pallas import tpu as pltpu
from jax.experimental.pallas import tpu_sc as plsc

B, T, C = 1, 2048, 2048
E, TOPK = 16, 2
H = 1024
HS = 2048
N = B * T
NA = N * TOPK
TM = 128
W = 48
P = W * TM
CH_ = C // 2

NC, NS = 2, 16
NW = NC * NS

_LOW = 0x0000FFFF
_HIGH = -65536


@functools.lru_cache(maxsize=None)
def _sc_mesh():
    return plsc.VectorSubcoreMesh(core_axis_name="c", subcore_axis_name="s")


def _pack_halves(left_f32, right_f32):
    lb = lax.bitcast_convert_type(
        left_f32.astype(jnp.bfloat16).astype(jnp.float32), jnp.int32)
    rb = lax.bitcast_convert_type(
        right_f32.astype(jnp.bfloat16).astype(jnp.float32), jnp.int32)
    return ((lb >> 16) & _LOW) | (rb & _HIGH)


def _unpack_halves(words_i32):
    left = lax.bitcast_convert_type(words_i32 << 16, jnp.float32)
    right = lax.bitcast_convert_type(words_i32 & _HIGH, jnp.float32)
    return left, right


def _pipelined_gathers(src_hbm, idx_v, out_hbm, out_base, bufs, sems, nch, ch):
    nbuf = len(bufs)
    gathers = [None] * nch
    writes = [None] * nch
    gathers[0] = pltpu.async_copy(
        src_hbm.at[idx_v.at[pl.ds(0, ch)]], bufs[0], sems[0])
    for ci in range(nch):
        k = ci % nbuf
        if ci + 1 < nch:
            kn = (ci + 1) % nbuf
            if ci + 1 >= nbuf:
                writes[ci + 1 - nbuf].wait()
            gathers[ci + 1] = pltpu.async_copy(
                src_hbm.at[idx_v.at[pl.ds((ci + 1) * ch, ch)]],
                bufs[kn], sems[kn])
        gathers[ci].wait()
        writes[ci] = pltpu.async_copy(
            bufs[k], out_hbm.at[pl.ds(out_base + ci * ch, ch)], sems[k])
    for ci in range(max(0, nch - nbuf), nch):
        writes[ci].wait()


def _router_body(x_ref, gw_ref, gb_ref, idx_ref, w_ref, xp_ref, cnt_ref,
                 carry_ref):
    t = pl.program_id(0)
    x = x_ref[...]
    s = jax.nn.sigmoid(
        jnp.dot(x, gw_ref[...], preferred_element_type=jnp.float32))
    sb = s + gb_ref[...]
    tm = s.shape[0]
    iota = lax.broadcasted_iota(jnp.int32, (tm, E), 1)
    m1 = jnp.max(sb, axis=1, keepdims=True)
    i1 = jnp.min(jnp.where(sb == m1, iota, E), axis=1, keepdims=True)
    sb2 = jnp.where(iota == i1, -jnp.inf, sb)
    m2 = jnp.max(sb2, axis=1, keepdims=True)
    i2 = jnp.min(jnp.where(sb2 == m2, iota, E), axis=1, keepdims=True)
    w1 = jnp.sum(jnp.where(iota == i1, s, 0.0), axis=1, keepdims=True)
    w2 = jnp.sum(jnp.where(iota == i2, s, 0.0), axis=1, keepdims=True)
    den = w1 + w2

    @pl.when(t == 0)
    def _():
        carry_ref[...] = jnp.zeros((1, E), jnp.float32)

    oh = ((iota == i1) | (iota == i2)).astype(jnp.float32)
    r = lax.broadcasted_iota(jnp.int32, (tm, tm), 0)
    c = lax.broadcasted_iota(jnp.int32, (tm, tm), 1)
    ltri = (r > c).astype(jnp.float32)
    prefix = (jnp.dot(ltri, oh, preferred_element_type=jnp.float32)
              + carry_ref[...])
    rank0 = jnp.sum(jnp.where(iota == i1, prefix, 0.0), axis=1, keepdims=True)
    rank1 = jnp.sum(jnp.where(iota == i2, prefix, 0.0), axis=1, keepdims=True)
    new_carry = carry_ref[...] + jnp.sum(oh, axis=0, keepdims=True)
    carry_ref[...] = new_carry
    cnt_ref[...] = new_carry.astype(jnp.int32)

    idx_ref[...] = jnp.where(
        iota == 0, i1,
        jnp.where(iota == 1, i2,
                  jnp.where(iota == 2, rank0.astype(jnp.int32),
                            jnp.where(iota == 3, rank1.astype(jnp.int32),
                                      0))))
    w_ref[...] = jnp.where(iota == 0, w1 / den,
                           jnp.where(iota == 1, w2 / den, 0.0))
    xp_ref[...] = _pack_halves(x[:, :CH_], x[:, CH_:])


def _router(x_flat, gate_w, gate_bias):
    tm = 256
    return pl.pallas_call(
        _router_body,
        grid=(N // tm,),
        in_specs=[
            pl.BlockSpec((tm, C), lambda t: (t, 0)),
            pl.BlockSpec((C, E), lambda t: (0, 0)),
            pl.BlockSpec((1, E), lambda t: (0, 0)),
        ],
        out_specs=[
            pl.BlockSpec((tm, E), lambda t: (t, 0)),
            pl.BlockSpec((tm, E), lambda t: (t, 0)),
            pl.BlockSpec((tm, CH_), lambda t: (t, 0)),
            pl.BlockSpec((1, E), lambda t: (0, 0)),
        ],
        out_shape=[
            jax.ShapeDtypeStruct((N, E), jnp.int32),
            jax.ShapeDtypeStruct((N, E), jnp.float32),
            jax.ShapeDtypeStruct((N, CH_), jnp.int32),
            jax.ShapeDtypeStruct((1, E), jnp.int32),
        ],
        scratch_shapes=[pltpu.VMEM((1, E), jnp.float32)],
    )(x_flat, gate_w, gate_bias.reshape(1, E))


_D_RPW = P // NW
_D_CH = 32
_D_NCH = _D_RPW // _D_CH


def _dispatch_body(x_hbm, tok_hbm, out_hbm, idx_v, b0, b1, s0, s1):
    wid = lax.axis_index("s") * NC + lax.axis_index("c")
    base = wid * _D_RPW
    pltpu.sync_copy(tok_hbm.at[pl.ds(base, _D_RPW)], idx_v)
    _pipelined_gathers(x_hbm, idx_v, out_hbm, base, (b0, b1), (s0, s1),
                       _D_NCH, _D_CH)


@functools.lru_cache(maxsize=None)
def _dispatch_kernel():
    return pl.kernel(
        _dispatch_body,
        out_type=jax.ShapeDtypeStruct((P, CH_), jnp.int32),
        mesh=_sc_mesh(),
        scratch_types=[
            pltpu.VMEM((_D_RPW,), jnp.int32),
            pltpu.VMEM((_D_CH, CH_), jnp.int32),
            pltpu.VMEM((_D_CH, CH_), jnp.int32),
            pltpu.SemaphoreType.DMA,
            pltpu.SemaphoreType.DMA,
        ],
    )


def _dispatch(x_packed, tok1d):
    return _dispatch_kernel()(x_packed, tok1d)


def _gemm_body(e_ref, v_ref, x_ref, uplo_ref, uphi_ref, dn_ref, w_ref, o_ref):
    i = pl.program_id(0)

    @pl.when(v_ref[i] == 1)
    def _():
        lo, hi = _unpack_halves(x_ref[...])
        yg = (jnp.dot(lo, uplo_ref[0], preferred_element_type=jnp.float32)
              + jnp.dot(hi, uphi_ref[0], preferred_element_type=jnp.float32))
        gv = yg[:, :H]
        uv = yg[:, H:]
        h = gv * jax.nn.sigmoid(gv) * uv
        oe = jnp.dot(h, dn_ref[0], preferred_element_type=jnp.float32)
        oe = oe * w_ref[:, 0:1]
        o_ref[...] = _pack_halves(oe[:, :CH_], oe[:, CH_:])


def _grouped_gemm(x_perm, up_w, down_w, wpad2d, e_for_tile, valid):
    grid_spec = pltpu.PrefetchScalarGridSpec(
        num_scalar_prefetch=2,
        grid=(W,),
        in_specs=[
            pl.BlockSpec((TM, CH_), lambda i, e, v: (i, 0)),
            pl.BlockSpec((1, CH_, 2 * H), lambda i, e, v: (e[i], 0, 0)),
            pl.BlockSpec((1, CH_, 2 * H), lambda i, e, v: (e[i], 1, 0)),
            pl.BlockSpec((1, H, C), lambda i, e, v: (e[i], 0, 0)),
            pl.BlockSpec((TM, 8), lambda i, e, v: (i, 0)),
        ],
        out_specs=pl.BlockSpec((TM, CH_), lambda i, e, v: (i, 0)),
    )
    return pl.pallas_call(
        _gemm_body,
        grid_spec=grid_spec,
        out_shape=jax.ShapeDtypeStruct((P, CH_), jnp.int32),
    )(e_for_tile, valid, x_perm, up_w, up_w, down_w, wpad2d)


_C_RPW = N // NW
_C_CH = 16
_C_NCH = _C_RPW // _C_CH


def _combine_body(src_hbm, p0_hbm, p1_hbm, g0_hbm, g1_hbm, i0_v, i1_v,
                  b00, b01, b10, b11, s00, s01, s10, s11):
    wid = lax.axis_index("s") * NC + lax.axis_index("c")
    base = wid * _C_RPW
    pltpu.sync_copy(p0_hbm.at[pl.ds(base, _C_RPW)], i0_v)
    pltpu.sync_copy(p1_hbm.at[pl.ds(base, _C_RPW)], i1_v)
    _pipelined_gathers(src_hbm, i0_v, g0_hbm, base, (b00, b01), (s00, s01),
                       _C_NCH, _C_CH)
    _pipelined_gathers(src_hbm, i1_v, g1_hbm, base, (b10, b11), (s10, s11),
                       _C_NCH, _C_CH)


@functools.lru_cache(maxsize=None)
def _combine_kernel():
    return pl.kernel(
        _combine_body,
        out_type=(
            jax.ShapeDtypeStruct((N, CH_), jnp.int32),
            jax.ShapeDtypeStruct((N, CH_), jnp.int32),
        ),
        mesh=_sc_mesh(),
        scratch_types=[
            pltpu.VMEM((_C_RPW,), jnp.int32),
            pltpu.VMEM((_C_RPW,), jnp.int32),
            pltpu.VMEM((_C_CH, CH_), jnp.int32),
            pltpu.VMEM((_C_CH, CH_), jnp.int32),
            pltpu.VMEM((_C_CH, CH_), jnp.int32),
            pltpu.VMEM((_C_CH, CH_), jnp.int32),
            pltpu.SemaphoreType.DMA,
            pltpu.SemaphoreType.DMA,
            pltpu.SemaphoreType.DMA,
            pltpu.SemaphoreType.DMA,
        ],
    )


def _combine(out_perm, pos0, pos1):
    return _combine_kernel()(out_perm, pos0, pos1)


def _shared_body(x_ref, sg_ref, sd_ref, g0_ref, g1_ref, o_ref):
    yg = jnp.dot(x_ref[...], sg_ref[...], preferred_element_type=jnp.float32)
    y = yg[:, :HS]
    gate = yg[:, HS:]
    h = gate * jax.nn.sigmoid(gate) * y
    base = jnp.dot(h, sd_ref[...], preferred_element_type=jnp.float32)
    g0l, g0r = _unpack_halves(g0_ref[...])
    g1l, g1r = _unpack_halves(g1_ref[...])
    o_ref[:, :CH_] = base[:, :CH_] + g0l + g1l
    o_ref[:, CH_:] = base[:, CH_:] + g0r + g1r


def _shared(x_flat, sgw, sdw, g0, g1):
    tm = 128
    return pl.pallas_call(
        _shared_body,
        grid=(N // tm,),
        in_specs=[
            pl.BlockSpec((tm, C), lambda t: (t, 0)),
            pl.BlockSpec((C, 2 * HS), lambda t: (0, 0)),
            pl.BlockSpec((HS, C), lambda t: (0, 0)),
            pl.BlockSpec((tm, CH_), lambda t: (t, 0)),
            pl.BlockSpec((tm, CH_), lambda t: (t, 0)),
        ],
        out_specs=pl.BlockSpec((tm, C), lambda t: (t, 0)),
        out_shape=jax.ShapeDtypeStruct((N, C), jnp.float32),
    )(x_flat, sgw, sdw, g0, g1)


def kernel(x, gate_w, gate_bias, shared_gate_w, shared_down_w, up_w, down_w):
    x_flat = x.reshape(N, C)

    idx_out, w_out, x_packed, cnt = _router(x_flat, gate_w, gate_bias)
    eN = idx_out[:, :TOPK].reshape(-1)
    wN = w_out[:, :TOPK].reshape(-1)
    rank = idx_out[:, TOPK:2 * TOPK].reshape(-1)
    counts = cnt[0]

    padded = ((counts + TM - 1) // TM) * TM
    pad_off = jnp.concatenate(
        [jnp.zeros((1,), jnp.int32), jnp.cumsum(padded).astype(jnp.int32)])
    ppos = pad_off[eN] + rank
    tok = (jnp.arange(NA, dtype=jnp.int32) // TOPK)
    tok_pad = (jnp.arange(P, dtype=jnp.int32) % N).at[ppos].set(
        tok, unique_indices=True)
    w_pad = jnp.zeros((P,), jnp.float32).at[ppos].set(
        wN, unique_indices=True)
    wpad2d = jnp.broadcast_to(w_pad[:, None], (P, 8))
    tile_off = pad_off // TM
    t_ar = jnp.arange(W, dtype=jnp.int32)
    e_for_tile = jnp.minimum(
        jnp.sum((t_ar[:, None] >= tile_off[None, 1:]).astype(jnp.int32),
                axis=1), E - 1).astype(jnp.int32)
    valid = (t_ar < tile_off[E]).astype(jnp.int32)
    pos0 = ppos[0::2]
    pos1 = ppos[1::2]

    x_perm = _dispatch(x_packed, tok_pad)

    out_perm = _grouped_gemm(x_perm, up_w, down_w, wpad2d, e_for_tile, valid)

    g0, g1 = _combine(out_perm, pos0, pos1)

    out = _shared(x_flat, shared_gate_w, shared_down_w, g0, g1)
    return out.reshape(B, T, C)


# speedup vs baseline: 3.2212x; 1.0700x over previous
import functools

import jax
import jax.numpy as jnp
from jax import lax
from jax.experimental import pallas as pl
from jax.experimental.pallas import tpu as pltpu
from jax.experimental.pallas import tpu_sc as plsc

B, T, C = 1, 2048, 2048
E, TOPK = 16, 2
H = 1024
HS = 2048
N = B * T
NA = N * TOPK
TM = 128
W = 48
P = W * TM
CH_ = C // 2

NC, NS = 2, 16
NW = NC * NS

_LOW = 0x0000FFFF
_HIGH = -65536


@functools.lru_cache(maxsize=None)
def _sc_mesh():
    return plsc.VectorSubcoreMesh(core_axis_name="c", subcore_axis_name="s")


def _pack_halves(left_f32, right_f32):
    lb = lax.bitcast_convert_type(
        left_f32.astype(jnp.bfloat16).astype(jnp.float32), jnp.int32)
    rb = lax.bitcast_convert_type(
        right_f32.astype(jnp.bfloat16).astype(jnp.float32), jnp.int32)
    return ((lb >> 16) & _LOW) | (rb & _HIGH)


def _unpack_halves(words_i32):
    left = lax.bitcast_convert_type(words_i32 << 16, jnp.float32)
    right = lax.bitcast_convert_type(words_i32 & _HIGH, jnp.float32)
    return left, right


def _pipelined_gathers(src_hbm, idx_v, out_hbm, out_base, bufs, sems, nch, ch):
    nbuf = len(bufs)
    gathers = [None] * nch
    writes = [None] * nch
    gathers[0] = pltpu.async_copy(
        src_hbm.at[idx_v.at[pl.ds(0, ch)]], bufs[0], sems[0])
    for ci in range(nch):
        k = ci % nbuf
        if ci + 1 < nch:
            kn = (ci + 1) % nbuf
            if ci + 1 >= nbuf:
                writes[ci + 1 - nbuf].wait()
            gathers[ci + 1] = pltpu.async_copy(
                src_hbm.at[idx_v.at[pl.ds((ci + 1) * ch, ch)]],
                bufs[kn], sems[kn])
        gathers[ci].wait()
        writes[ci] = pltpu.async_copy(
            bufs[k], out_hbm.at[pl.ds(out_base + ci * ch, ch)], sems[k])
    for ci in range(max(0, nch - nbuf), nch):
        writes[ci].wait()


def _router_body(x_ref, gw_ref, gb_ref, idx_ref, w_ref, xp_ref, cnt_ref,
                 carry_ref):
    t = pl.program_id(0)
    x = x_ref[...]
    s = jax.nn.sigmoid(
        jnp.dot(x, gw_ref[...], preferred_element_type=jnp.float32))
    sb = s + gb_ref[...]
    tm = s.shape[0]
    iota = lax.broadcasted_iota(jnp.int32, (tm, E), 1)
    m1 = jnp.max(sb, axis=1, keepdims=True)
    i1 = jnp.min(jnp.where(sb == m1, iota, E), axis=1, keepdims=True)
    sb2 = jnp.where(iota == i1, -jnp.inf, sb)
    m2 = jnp.max(sb2, axis=1, keepdims=True)
    i2 = jnp.min(jnp.where(sb2 == m2, iota, E), axis=1, keepdims=True)
    w1 = jnp.sum(jnp.where(iota == i1, s, 0.0), axis=1, keepdims=True)
    w2 = jnp.sum(jnp.where(iota == i2, s, 0.0), axis=1, keepdims=True)
    den = w1 + w2

    @pl.when(t == 0)
    def _():
        carry_ref[...] = jnp.zeros((1, E), jnp.float32)

    oh = ((iota == i1) | (iota == i2)).astype(jnp.float32)
    r = lax.broadcasted_iota(jnp.int32, (tm, tm), 0)
    c = lax.broadcasted_iota(jnp.int32, (tm, tm), 1)
    ltri = (r > c).astype(jnp.float32)
    prefix = (jnp.dot(ltri, oh, preferred_element_type=jnp.float32)
              + carry_ref[...])
    rank0 = jnp.sum(jnp.where(iota == i1, prefix, 0.0), axis=1, keepdims=True)
    rank1 = jnp.sum(jnp.where(iota == i2, prefix, 0.0), axis=1, keepdims=True)
    new_carry = carry_ref[...] + jnp.sum(oh, axis=0, keepdims=True)
    carry_ref[...] = new_carry
    cnt_ref[...] = new_carry.astype(jnp.int32)

    idx_ref[...] = jnp.where(
        iota == 0, i1,
        jnp.where(iota == 1, i2,
                  jnp.where(iota == 2, rank0.astype(jnp.int32),
                            jnp.where(iota == 3, rank1.astype(jnp.int32),
                                      0))))
    w_ref[...] = jnp.where(iota == 0, w1 / den,
                           jnp.where(iota == 1, w2 / den, 0.0))
    xp_ref[...] = _pack_halves(x[:, :CH_], x[:, CH_:])


def _router(x_flat, gate_w, gate_bias):
    tm = 256
    return pl.pallas_call(
        _router_body,
        grid=(N // tm,),
        in_specs=[
            pl.BlockSpec((tm, C), lambda t: (t, 0)),
            pl.BlockSpec((C, E), lambda t: (0, 0)),
            pl.BlockSpec((1, E), lambda t: (0, 0)),
        ],
        out_specs=[
            pl.BlockSpec((tm, E), lambda t: (t, 0)),
            pl.BlockSpec((tm, E), lambda t: (t, 0)),
            pl.BlockSpec((tm, CH_), lambda t: (t, 0)),
            pl.BlockSpec((1, E), lambda t: (0, 0)),
        ],
        out_shape=[
            jax.ShapeDtypeStruct((N, E), jnp.int32),
            jax.ShapeDtypeStruct((N, E), jnp.float32),
            jax.ShapeDtypeStruct((N, CH_), jnp.int32),
            jax.ShapeDtypeStruct((1, E), jnp.int32),
        ],
        scratch_shapes=[pltpu.VMEM((1, E), jnp.float32)],
    )(x_flat, gate_w, gate_bias.reshape(1, E))


_D_TPW = N // NW


def _dispatch_body(x_hbm, p0_hbm, p1_hbm, out_hbm, i0_v, i1_v, buf, s0, s1):
    wid = lax.axis_index("s") * NC + lax.axis_index("c")
    tbase = wid * _D_TPW
    pltpu.sync_copy(p0_hbm.at[pl.ds(tbase, _D_TPW)], i0_v)
    pltpu.sync_copy(p1_hbm.at[pl.ds(tbase, _D_TPW)], i1_v)
    pltpu.sync_copy(x_hbm.at[pl.ds(tbase, _D_TPW)], buf)
    c0 = pltpu.async_copy(buf, out_hbm.at[i0_v], s0)
    c1 = pltpu.async_copy(buf, out_hbm.at[i1_v], s1)
    c0.wait()
    c1.wait()


@functools.lru_cache(maxsize=None)
def _dispatch_kernel():
    return pl.kernel(
        _dispatch_body,
        out_type=jax.ShapeDtypeStruct((P, CH_), jnp.int32),
        mesh=_sc_mesh(),
        scratch_types=[
            pltpu.VMEM((_D_TPW,), jnp.int32),
            pltpu.VMEM((_D_TPW,), jnp.int32),
            pltpu.VMEM((_D_TPW, CH_), jnp.int32),
            pltpu.SemaphoreType.DMA,
            pltpu.SemaphoreType.DMA,
        ],
    )


def _dispatch(x_packed, pos0, pos1):
    return _dispatch_kernel()(x_packed, pos0, pos1)


def _gemm_body(e_ref, v_ref, x_ref, uplo_ref, uphi_ref, dn_ref, o_ref):
    i = pl.program_id(0)

    @pl.when(v_ref[i] == 1)
    def _():
        lo, hi = _unpack_halves(x_ref[...])
        yg = (jnp.dot(lo, uplo_ref[0], preferred_element_type=jnp.float32)
              + jnp.dot(hi, uphi_ref[0], preferred_element_type=jnp.float32))
        gv = yg[:, :H]
        uv = yg[:, H:]
        h = gv * jax.nn.sigmoid(gv) * uv
        oe = jnp.dot(h, dn_ref[0], preferred_element_type=jnp.float32)
        o_ref[...] = _pack_halves(oe[:, :CH_], oe[:, CH_:])


def _grouped_gemm(x_perm, up_w, down_w, e_for_tile, valid):
    grid_spec = pltpu.PrefetchScalarGridSpec(
        num_scalar_prefetch=2,
        grid=(W,),
        in_specs=[
            pl.BlockSpec((TM, CH_), lambda i, e, v: (i, 0)),
            pl.BlockSpec((1, CH_, 2 * H), lambda i, e, v: (e[i], 0, 0)),
            pl.BlockSpec((1, CH_, 2 * H), lambda i, e, v: (e[i], 1, 0)),
            pl.BlockSpec((1, H, C), lambda i, e, v: (e[i], 0, 0)),
        ],
        out_specs=pl.BlockSpec((TM, CH_), lambda i, e, v: (i, 0)),
    )
    return pl.pallas_call(
        _gemm_body,
        grid_spec=grid_spec,
        out_shape=jax.ShapeDtypeStruct((P, CH_), jnp.int32),
    )(e_for_tile, valid, x_perm, up_w, up_w, down_w)


_C_RPW = N // NW
_C_CH = 16
_C_NCH = _C_RPW // _C_CH


def _combine_body(src_hbm, p0_hbm, p1_hbm, g0_hbm, g1_hbm, i0_v, i1_v,
                  b00, b01, b10, b11, s00, s01, s10, s11):
    wid = lax.axis_index("s") * NC + lax.axis_index("c")
    base = wid * _C_RPW
    pltpu.sync_copy(p0_hbm.at[pl.ds(base, _C_RPW)], i0_v)
    pltpu.sync_copy(p1_hbm.at[pl.ds(base, _C_RPW)], i1_v)
    _pipelined_gathers(src_hbm, i0_v, g0_hbm, base, (b00, b01), (s00, s01),
                       _C_NCH, _C_CH)
    _pipelined_gathers(src_hbm, i1_v, g1_hbm, base, (b10, b11), (s10, s11),
                       _C_NCH, _C_CH)


@functools.lru_cache(maxsize=None)
def _combine_kernel():
    return pl.kernel(
        _combine_body,
        out_type=(
            jax.ShapeDtypeStruct((N, CH_), jnp.int32),
            jax.ShapeDtypeStruct((N, CH_), jnp.int32),
        ),
        mesh=_sc_mesh(),
        scratch_types=[
            pltpu.VMEM((_C_RPW,), jnp.int32),
            pltpu.VMEM((_C_RPW,), jnp.int32),
            pltpu.VMEM((_C_CH, CH_), jnp.int32),
            pltpu.VMEM((_C_CH, CH_), jnp.int32),
            pltpu.VMEM((_C_CH, CH_), jnp.int32),
            pltpu.VMEM((_C_CH, CH_), jnp.int32),
            pltpu.SemaphoreType.DMA,
            pltpu.SemaphoreType.DMA,
            pltpu.SemaphoreType.DMA,
            pltpu.SemaphoreType.DMA,
        ],
    )


def _combine(out_perm, pos0, pos1):
    return _combine_kernel()(out_perm, pos0, pos1)


def _shared_body(x_ref, sg_ref, sd_ref, g0_ref, g1_ref, w_ref, o_ref):
    yg = jnp.dot(x_ref[...], sg_ref[...], preferred_element_type=jnp.float32)
    y = yg[:, :HS]
    gate = yg[:, HS:]
    h = gate * jax.nn.sigmoid(gate) * y
    base = jnp.dot(h, sd_ref[...], preferred_element_type=jnp.float32)
    w0 = w_ref[:, 0:1]
    w1 = w_ref[:, 1:2]
    g0l, g0r = _unpack_halves(g0_ref[...])
    g1l, g1r = _unpack_halves(g1_ref[...])
    o_ref[:, :CH_] = base[:, :CH_] + w0 * g0l + w1 * g1l
    o_ref[:, CH_:] = base[:, CH_:] + w0 * g0r + w1 * g1r


def _shared(x_flat, sgw, sdw, g0, g1, w_out):
    tm = 128
    return pl.pallas_call(
        _shared_body,
        grid=(N // tm,),
        in_specs=[
            pl.BlockSpec((tm, C), lambda t: (t, 0)),
            pl.BlockSpec((C, 2 * HS), lambda t: (0, 0)),
            pl.BlockSpec((HS, C), lambda t: (0, 0)),
            pl.BlockSpec((tm, CH_), lambda t: (t, 0)),
            pl.BlockSpec((tm, CH_), lambda t: (t, 0)),
            pl.BlockSpec((tm, E), lambda t: (t, 0)),
        ],
        out_specs=pl.BlockSpec((tm, C), lambda t: (t, 0)),
        out_shape=jax.ShapeDtypeStruct((N, C), jnp.float32),
    )(x_flat, sgw, sdw, g0, g1, w_out)


def kernel(x, gate_w, gate_bias, shared_gate_w, shared_down_w, up_w, down_w):
    x_flat = x.reshape(N, C)

    idx_out, w_out, x_packed, cnt = _router(x_flat, gate_w, gate_bias)
    eN = idx_out[:, :TOPK].reshape(-1)
    wN = w_out[:, :TOPK].reshape(-1)
    rank = idx_out[:, TOPK:2 * TOPK].reshape(-1)
    counts = cnt[0]

    padded = ((counts + TM - 1) // TM) * TM
    pad_off = jnp.concatenate(
        [jnp.zeros((1,), jnp.int32), jnp.cumsum(padded).astype(jnp.int32)])
    ppos = pad_off[eN] + rank
    tile_off = pad_off // TM
    t_ar = jnp.arange(W, dtype=jnp.int32)
    e_for_tile = jnp.minimum(
        jnp.sum((t_ar[:, None] >= tile_off[None, 1:]).astype(jnp.int32),
                axis=1), E - 1).astype(jnp.int32)
    valid = (t_ar < tile_off[E]).astype(jnp.int32)
    pos0 = ppos[0::2]
    pos1 = ppos[1::2]

    x_perm = _dispatch(x_packed, pos0, pos1)

    out_perm = _grouped_gemm(x_perm, up_w, down_w, e_for_tile, valid)

    g0, g1 = _combine(out_perm, pos0, pos1)

    out = _shared(x_flat, shared_gate_w, shared_down_w, g0, g1, w_out)
    return out.reshape(B, T, C)


# speedup vs baseline: 3.2730x; 1.0161x over previous
import functools

import jax
import jax.numpy as jnp
from jax import lax
from jax.experimental import pallas as pl
from jax.experimental.pallas import tpu as pltpu
from jax.experimental.pallas import tpu_sc as plsc

B, T, C = 1, 2048, 2048
E, TOPK = 16, 2
H = 1024
HS = 2048
N = B * T
NA = N * TOPK
TM = 128
W = 48
P = W * TM
CH_ = C // 2

NC, NS = 2, 16
NW = NC * NS

_LOW = 0x0000FFFF
_HIGH = -65536


@functools.lru_cache(maxsize=None)
def _sc_mesh():
    return plsc.VectorSubcoreMesh(core_axis_name="c", subcore_axis_name="s")


def _pack_halves(left_f32, right_f32):
    lb = lax.bitcast_convert_type(
        left_f32.astype(jnp.bfloat16).astype(jnp.float32), jnp.int32)
    rb = lax.bitcast_convert_type(
        right_f32.astype(jnp.bfloat16).astype(jnp.float32), jnp.int32)
    return ((lb >> 16) & _LOW) | (rb & _HIGH)


def _unpack_halves(words_i32):
    left = lax.bitcast_convert_type(words_i32 << 16, jnp.float32)
    right = lax.bitcast_convert_type(words_i32 & _HIGH, jnp.float32)
    return left, right


def _pipelined_gathers(src_hbm, idx_v, out_hbm, out_base, bufs, sems, nch, ch):
    nbuf = len(bufs)
    gathers = [None] * nch
    writes = [None] * nch
    gathers[0] = pltpu.async_copy(
        src_hbm.at[idx_v.at[pl.ds(0, ch)]], bufs[0], sems[0])
    for ci in range(nch):
        k = ci % nbuf
        if ci + 1 < nch:
            kn = (ci + 1) % nbuf
            if ci + 1 >= nbuf:
                writes[ci + 1 - nbuf].wait()
            gathers[ci + 1] = pltpu.async_copy(
                src_hbm.at[idx_v.at[pl.ds((ci + 1) * ch, ch)]],
                bufs[kn], sems[kn])
        gathers[ci].wait()
        writes[ci] = pltpu.async_copy(
            bufs[k], out_hbm.at[pl.ds(out_base + ci * ch, ch)], sems[k])
    for ci in range(max(0, nch - nbuf), nch):
        writes[ci].wait()


def _router_body(x_ref, gw_ref, gb_ref, idx_ref, w_ref, xp_ref, cnt_ref,
                 carry_ref):
    t = pl.program_id(0)
    x = x_ref[...]
    s = jax.nn.sigmoid(
        jnp.dot(x, gw_ref[...], preferred_element_type=jnp.float32))
    sb = s + gb_ref[...]
    tm = s.shape[0]
    iota = lax.broadcasted_iota(jnp.int32, (tm, E), 1)
    m1 = jnp.max(sb, axis=1, keepdims=True)
    i1 = jnp.min(jnp.where(sb == m1, iota, E), axis=1, keepdims=True)
    sb2 = jnp.where(iota == i1, -jnp.inf, sb)
    m2 = jnp.max(sb2, axis=1, keepdims=True)
    i2 = jnp.min(jnp.where(sb2 == m2, iota, E), axis=1, keepdims=True)
    w1 = jnp.sum(jnp.where(iota == i1, s, 0.0), axis=1, keepdims=True)
    w2 = jnp.sum(jnp.where(iota == i2, s, 0.0), axis=1, keepdims=True)
    den = w1 + w2

    @pl.when(t == 0)
    def _():
        carry_ref[...] = jnp.zeros((1, E), jnp.float32)

    oh = ((iota == i1) | (iota == i2)).astype(jnp.float32)
    r = lax.broadcasted_iota(jnp.int32, (tm, tm), 0)
    c = lax.broadcasted_iota(jnp.int32, (tm, tm), 1)
    ltri = (r > c).astype(jnp.float32)
    prefix = (jnp.dot(ltri, oh, preferred_element_type=jnp.float32)
              + carry_ref[...])
    rank0 = jnp.sum(jnp.where(iota == i1, prefix, 0.0), axis=1, keepdims=True)
    rank1 = jnp.sum(jnp.where(iota == i2, prefix, 0.0), axis=1, keepdims=True)
    new_carry = carry_ref[...] + jnp.sum(oh, axis=0, keepdims=True)
    carry_ref[...] = new_carry
    cnt_ref[...] = new_carry.astype(jnp.int32)

    idx_ref[...] = jnp.where(
        iota == 0, i1,
        jnp.where(iota == 1, i2,
                  jnp.where(iota == 2, rank0.astype(jnp.int32),
                            jnp.where(iota == 3, rank1.astype(jnp.int32),
                                      0))))
    w_ref[...] = jnp.where(iota == 0, w1 / den,
                           jnp.where(iota == 1, w2 / den, 0.0))
    xp_ref[...] = _pack_halves(x[:, :CH_], x[:, CH_:])


def _router(x_flat, gate_w, gate_bias):
    tm = 256
    return pl.pallas_call(
        _router_body,
        grid=(N // tm,),
        in_specs=[
            pl.BlockSpec((tm, C), lambda t: (t, 0)),
            pl.BlockSpec((C, E), lambda t: (0, 0)),
            pl.BlockSpec((1, E), lambda t: (0, 0)),
        ],
        out_specs=[
            pl.BlockSpec((tm, E), lambda t: (t, 0)),
            pl.BlockSpec((tm, E), lambda t: (t, 0)),
            pl.BlockSpec((tm, CH_), lambda t: (t, 0)),
            pl.BlockSpec((1, E), lambda t: (0, 0)),
        ],
        out_shape=[
            jax.ShapeDtypeStruct((N, E), jnp.int32),
            jax.ShapeDtypeStruct((N, E), jnp.float32),
            jax.ShapeDtypeStruct((N, CH_), jnp.int32),
            jax.ShapeDtypeStruct((1, E), jnp.int32),
        ],
        scratch_shapes=[pltpu.VMEM((1, E), jnp.float32)],
    )(x_flat, gate_w, gate_bias.reshape(1, E))


_D_TPW = N // NW


def _dispatch_body(x_hbm, p0_hbm, p1_hbm, out_hbm, i0_v, i1_v, buf, s0, s1):
    wid = lax.axis_index("s") * NC + lax.axis_index("c")
    tbase = wid * _D_TPW
    pltpu.sync_copy(p0_hbm.at[pl.ds(tbase, _D_TPW)], i0_v)
    pltpu.sync_copy(p1_hbm.at[pl.ds(tbase, _D_TPW)], i1_v)
    pltpu.sync_copy(x_hbm.at[pl.ds(tbase, _D_TPW)], buf)
    c0 = pltpu.async_copy(buf, out_hbm.at[i0_v], s0)
    c1 = pltpu.async_copy(buf, out_hbm.at[i1_v], s1)
    c0.wait()
    c1.wait()


@functools.lru_cache(maxsize=None)
def _dispatch_kernel():
    return pl.kernel(
        _dispatch_body,
        out_type=jax.ShapeDtypeStruct((P, CH_), jnp.int32),
        mesh=_sc_mesh(),
        scratch_types=[
            pltpu.VMEM((_D_TPW,), jnp.int32),
            pltpu.VMEM((_D_TPW,), jnp.int32),
            pltpu.VMEM((_D_TPW, CH_), jnp.int32),
            pltpu.SemaphoreType.DMA,
            pltpu.SemaphoreType.DMA,
        ],
    )


def _dispatch(x_packed, pos0, pos1):
    return _dispatch_kernel()(x_packed, pos0, pos1)


def _gemm_body(e_ref, v_ref, tc_ref, x_ref, uplo_ref, uphi_ref, dn_ref,
               o_ref):
    i = pl.program_id(0)

    @pl.when(v_ref[i] == 1)
    def _():
        lo, hi = _unpack_halves(x_ref[...])
        yg = (jnp.dot(lo, uplo_ref[0], preferred_element_type=jnp.float32)
              + jnp.dot(hi, uphi_ref[0], preferred_element_type=jnp.float32))
        gv = yg[:, :H]
        uv = yg[:, H:]
        h = gv * jax.nn.sigmoid(gv) * uv
        oe = jnp.dot(h, dn_ref[0], preferred_element_type=jnp.float32)
        o_ref[...] = _pack_halves(oe[:, :CH_], oe[:, CH_:])


def _grouped_gemm(x_perm, up_w, down_w, e_for_tile, valid, t_clamp):
    grid_spec = pltpu.PrefetchScalarGridSpec(
        num_scalar_prefetch=3,
        grid=(W,),
        in_specs=[
            pl.BlockSpec((TM, CH_), lambda i, e, v, tc: (tc[i], 0)),
            pl.BlockSpec((1, CH_, 2 * H), lambda i, e, v, tc: (e[i], 0, 0)),
            pl.BlockSpec((1, CH_, 2 * H), lambda i, e, v, tc: (e[i], 1, 0)),
            pl.BlockSpec((1, H, C), lambda i, e, v, tc: (e[i], 0, 0)),
        ],
        out_specs=pl.BlockSpec((TM, CH_), lambda i, e, v, tc: (tc[i], 0)),
    )
    return pl.pallas_call(
        _gemm_body,
        grid_spec=grid_spec,
        out_shape=jax.ShapeDtypeStruct((P, CH_), jnp.int32),
    )(e_for_tile, valid, t_clamp, x_perm, up_w, up_w, down_w)


_C_RPW = N // NW
_C_CH = 16
_C_NCH = _C_RPW // _C_CH


def _combine_body(src_hbm, p0_hbm, p1_hbm, g0_hbm, g1_hbm, i0_v, i1_v,
                  b00, b01, b10, b11, s00, s01, s10, s11):
    wid = lax.axis_index("s") * NC + lax.axis_index("c")
    base = wid * _C_RPW
    pltpu.sync_copy(p0_hbm.at[pl.ds(base, _C_RPW)], i0_v)
    pltpu.sync_copy(p1_hbm.at[pl.ds(base, _C_RPW)], i1_v)
    _pipelined_gathers(src_hbm, i0_v, g0_hbm, base, (b00, b01), (s00, s01),
                       _C_NCH, _C_CH)
    _pipelined_gathers(src_hbm, i1_v, g1_hbm, base, (b10, b11), (s10, s11),
                       _C_NCH, _C_CH)


@functools.lru_cache(maxsize=None)
def _combine_kernel():
    return pl.kernel(
        _combine_body,
        out_type=(
            jax.ShapeDtypeStruct((N, CH_), jnp.int32),
            jax.ShapeDtypeStruct((N, CH_), jnp.int32),
        ),
        mesh=_sc_mesh(),
        scratch_types=[
            pltpu.VMEM((_C_RPW,), jnp.int32),
            pltpu.VMEM((_C_RPW,), jnp.int32),
            pltpu.VMEM((_C_CH, CH_), jnp.int32),
            pltpu.VMEM((_C_CH, CH_), jnp.int32),
            pltpu.VMEM((_C_CH, CH_), jnp.int32),
            pltpu.VMEM((_C_CH, CH_), jnp.int32),
            pltpu.SemaphoreType.DMA,
            pltpu.SemaphoreType.DMA,
            pltpu.SemaphoreType.DMA,
            pltpu.SemaphoreType.DMA,
        ],
    )


def _combine(out_perm, pos0, pos1):
    return _combine_kernel()(out_perm, pos0, pos1)


def _shared_body(x_ref, sg_ref, sd_ref, g0_ref, g1_ref, w_ref, o_ref):
    yg = jnp.dot(x_ref[...], sg_ref[...], preferred_element_type=jnp.float32)
    y = yg[:, :HS]
    gate = yg[:, HS:]
    h = gate * jax.nn.sigmoid(gate) * y
    base = jnp.dot(h, sd_ref[...], preferred_element_type=jnp.float32)
    w0 = w_ref[:, 0:1]
    w1 = w_ref[:, 1:2]
    g0l, g0r = _unpack_halves(g0_ref[...])
    g1l, g1r = _unpack_halves(g1_ref[...])
    o_ref[:, :CH_] = base[:, :CH_] + w0 * g0l + w1 * g1l
    o_ref[:, CH_:] = base[:, CH_:] + w0 * g0r + w1 * g1r


def _shared(x_flat, sgw, sdw, g0, g1, w_out):
    tm = 128
    return pl.pallas_call(
        _shared_body,
        grid=(N // tm,),
        in_specs=[
            pl.BlockSpec((tm, C), lambda t: (t, 0)),
            pl.BlockSpec((C, 2 * HS), lambda t: (0, 0)),
            pl.BlockSpec((HS, C), lambda t: (0, 0)),
            pl.BlockSpec((tm, CH_), lambda t: (t, 0)),
            pl.BlockSpec((tm, CH_), lambda t: (t, 0)),
            pl.BlockSpec((tm, E), lambda t: (t, 0)),
        ],
        out_specs=pl.BlockSpec((tm, C), lambda t: (t, 0)),
        out_shape=jax.ShapeDtypeStruct((N, C), jnp.float32),
    )(x_flat, sgw, sdw, g0, g1, w_out)


def kernel(x, gate_w, gate_bias, shared_gate_w, shared_down_w, up_w, down_w):
    x_flat = x.reshape(N, C)

    idx_out, w_out, x_packed, cnt = _router(x_flat, gate_w, gate_bias)
    eN = idx_out[:, :TOPK].reshape(-1)
    wN = w_out[:, :TOPK].reshape(-1)
    rank = idx_out[:, TOPK:2 * TOPK].reshape(-1)
    counts = cnt[0]

    padded = ((counts + TM - 1) // TM) * TM
    pad_off = jnp.concatenate(
        [jnp.zeros((1,), jnp.int32), jnp.cumsum(padded).astype(jnp.int32)])
    ppos = pad_off[eN] + rank
    tile_off = pad_off // TM
    t_ar = jnp.arange(W, dtype=jnp.int32)
    e_for_tile = jnp.minimum(
        jnp.sum((t_ar[:, None] >= tile_off[None, 1:]).astype(jnp.int32),
                axis=1), E - 1).astype(jnp.int32)
    valid = (t_ar < tile_off[E]).astype(jnp.int32)
    t_clamp = jnp.minimum(t_ar, tile_off[E] - 1).astype(jnp.int32)
    pos0 = ppos[0::2]
    pos1 = ppos[1::2]

    x_perm = _dispatch(x_packed, pos0, pos1)

    out_perm = _grouped_gemm(x_perm, up_w, down_w, e_for_tile, valid,
                             t_clamp)

    g0, g1 = _combine(out_perm, pos0, pos1)

    out = _shared(x_flat, shared_gate_w, shared_down_w, g0, g1, w_out)
    return out.reshape(B, T, C)
